# baseline (speedup 1.0000x reference)
import jax
import jax.numpy as jnp
from jax import lax
from jax.experimental import pallas as pl
from jax.experimental.pallas import tpu as pltpu
from jax.experimental.pallas import tpu_sc as plsc

_N = 10000
_E = 320000
_R = 8
_H = 64
_C = 16
_RN = _R * _N
_NCH = 19
_SCH = 4352
_ACC_H = 4480
_TRASH = 4400
_PADSEG = 80500
_NS = 16
_BLK = 128
_KB = 158
_KBP = 160
_EPT = _KB * _BLK
_EPAD = _NS * _EPT
_PADROW = _EPAD // _BLK - 1
_ZPT = _ACC_H // _NS
_BR = 1000
_NPAD = 20480


def _make_sc_agg():
    mesh = plsc.VectorSubcoreMesh(
        core_axis_name="c", subcore_axis_name="s", num_cores=1)
    outs = [jax.ShapeDtypeStruct((_RN, 128), jnp.float32)]
    scratch = [
        pltpu.VMEM((_KBP, _BLK), jnp.int32),
        pltpu.VMEM((_KBP, _BLK), jnp.int32),
        pltpu.VMEM((_KBP, _BLK), jnp.int32),
        pltpu.VMEM((2, _BLK), jnp.int32),
        pltpu.VMEM((2, _BLK), jnp.int32),
        pltpu.VMEM((_BLK, 128), jnp.float32),
        pltpu.VMEM((_BLK, 128), jnp.float32),
        pltpu.VMEM_SHARED((_ACC_H, 128), jnp.float32),
        pltpu.SemaphoreType.DMA,
    ]

    def body(table_hbm, pk, a_out, *rest):
        pkv, srcv, sidx_v, idxp, idxf, buf, zbuf, acc, sem = rest
        s = lax.axis_index("s")

        def lanes(g):
            return lax.iota(jnp.int32, 16) + g * 16

        for g in range(8):
            idxp[0, pl.ds(g * 16, 16)] = s * _KB + lanes(g)
            w = s * _KB + 128 + lanes(g)
            idxp[1, pl.ds(g * 16, 16)] = jnp.where(
                lanes(g) < _KB - 128, w, _PADROW)
        pltpu.async_copy(pk.at[idxp.at[0]], pkv.at[pl.ds(0, 128)], sem).wait()
        pltpu.async_copy(pk.at[idxp.at[1, pl.ds(0, 32)]],
                         pkv.at[pl.ds(128, 32)], sem).wait()

        def upk(j, carry):
            for k in range(_BLK // 16):
                w = pkv[j, pl.ds(k * 16, 16)]
                srcv[j, pl.ds(k * 16, 16)] = lax.shift_right_logical(w, 17)
            return carry
        lax.fori_loop(0, _KBP, upk, 0)

        def zf(i, carry):
            for k in range(8):
                zbuf[i, pl.ds(k * 16, 16)] = jnp.zeros((16,), jnp.float32)
            return carry
        lax.fori_loop(0, _BLK, zf, 0)

        zstarts = [0, 128, _ZPT - 128]

        def zero_acc():
            for st in zstarts:
                for g in range(8):
                    idxf[0, pl.ds(g * 16, 16)] = s * _ZPT + st + lanes(g)
                pltpu.sync_copy(zbuf, acc.at[idxf.at[0]])
        zero_acc()

        def flush(dst, ch):
            ngroups = jnp.where(ch < _NCH - 1, 34, 13)
            for g in range(3):
                gid = s + 16 * g

                @pl.when(gid < ngroups)
                def _():
                    for q in range(8):
                        idxf[0, pl.ds(q * 16, 16)] = gid * 128 + lanes(q)
                        idxf[1, pl.ds(q * 16, 16)] = \
                            ch * _SCH + gid * 128 + lanes(q)
                    pltpu.async_copy(acc.at[idxf.at[0]], buf, sem).wait()
                    pltpu.sync_copy(buf, dst.at[idxf.at[1]])

        def sweep(add_block, dst):
            def chunk_body(ch, carry):
                def cs(j, carry2):
                    for k in range(_BLK // 16):
                        w = pkv[j, pl.ds(k * 16, 16)]
                        loc = lax.bitwise_and(w, 0x1FFFF) - ch * _SCH
                        ok = (loc >= 0) & (loc < _SCH)
                        sidx_v[j, pl.ds(k * 16, 16)] = jnp.where(
                            ok, loc, _TRASH)
                    return carry2
                lax.fori_loop(0, _KBP, cs, 0)
                plsc.subcore_barrier()

                def eb(j, carry2):
                    add_block(j)
                    return carry2
                lax.fori_loop(0, _KBP, eb, 0)
                plsc.subcore_barrier()
                flush(dst, ch)
                plsc.subcore_barrier()
                zero_acc()
                return carry
            lax.fori_loop(0, _NCH, chunk_body, 0)

        def ab(j):
            pltpu.async_copy(table_hbm.at[srcv.at[j]], buf, sem).wait()
            pltpu.sync_copy(buf, acc.at[sidx_v.at[j]], add=True)
        sweep(ab, a_out)

    return pl.kernel(body, mesh=mesh, out_type=outs, scratch_types=scratch)


def _make_layer(fin, has_bn):
    def body(*refs):
        if has_bn:
            (h, a, cnt, w, root, b, sums_in, g, be, z_out, sums_out) = refs
        else:
            (h, a, cnt, w, root, b, z_out, sums_out) = refs
        i = pl.program_id(0)
        hb = h[...][:, :fin]
        if has_bn:
            mu = sums_in[0, :] / _N
            var = sums_in[1, :] / _N - mu * mu
            isd = lax.rsqrt(var + 1e-5)
            alpha = g[0, :] * isd
            beta = be[0, :] - mu * alpha
            hb = hb * alpha[None, :] + beta[None, :]
        acc = jnp.dot(hb, root[...], preferred_element_type=jnp.float32)
        acc = acc + b[0, :][None, :]
        for r in range(_R):
            cb = cnt[r][:, 0:1]
            ar = a[r][:, :fin] * (1.0 / jnp.maximum(cb, 1.0))
            if has_bn:
                ar = ar * alpha[None, :] \
                    + (cb > 0.0).astype(jnp.float32) * beta[None, :]
            acc = acc + jnp.dot(ar, w[r], preferred_element_type=jnp.float32)
        zv = jnp.maximum(acc, 0.0)
        z_out[...] = jnp.concatenate(
            [zv, jnp.zeros((_BR, 128 - _H), jnp.float32)], axis=1)

        @pl.when(i == 0)
        def _():
            sums_out[...] = jnp.zeros((8, _H), jnp.float32)
        sums_out[0, :] = sums_out[0, :] + jnp.sum(zv, axis=0)
        sums_out[1, :] = sums_out[1, :] + jnp.sum(zv * zv, axis=0)

    in_specs = [
        pl.BlockSpec((_BR, 128), lambda i: (i, 0)),
        pl.BlockSpec((_R, _BR, 128), lambda i: (0, i, 0)),
        pl.BlockSpec((_R, _BR, 128), lambda i: (0, i, 0)),
        pl.BlockSpec((_R, fin, _H), lambda i: (0, 0, 0)),
        pl.BlockSpec((fin, _H), lambda i: (0, 0)),
        pl.BlockSpec((1, _H), lambda i: (0, 0)),
    ]
    if has_bn:
        in_specs += [
            pl.BlockSpec((8, _H), lambda i: (0, 0)),
            pl.BlockSpec((1, _H), lambda i: (0, 0)),
            pl.BlockSpec((1, _H), lambda i: (0, 0)),
        ]
    return pl.pallas_call(
        body,
        grid=(_N // _BR,),
        in_specs=in_specs,
        out_specs=[
            pl.BlockSpec((_BR, 128), lambda i: (i, 0)),
            pl.BlockSpec((8, _H), lambda i: (0, 0)),
        ],
        out_shape=[
            jax.ShapeDtypeStruct((_NPAD, 128), jnp.float32),
            jax.ShapeDtypeStruct((8, _H), jnp.float32),
        ],
    )


def _make_final():
    def body(z, sums, g, be, wl, bl, out):
        mu = sums[0, :] / _N
        var = sums[1, :] / _N - mu * mu
        isd = lax.rsqrt(var + 1e-5)
        alpha = g[0, :] * isd
        beta = be[0, :] - mu * alpha
        hb = z[...][:, :_H] * alpha[None, :] + beta[None, :]
        out[...] = jnp.dot(hb, wl[...], preferred_element_type=jnp.float32) \
            + bl[0, :][None, :]

    return pl.pallas_call(
        body,
        grid=(_N // _BR,),
        in_specs=[
            pl.BlockSpec((_BR, 128), lambda i: (i, 0)),
            pl.BlockSpec((8, _H), lambda i: (0, 0)),
            pl.BlockSpec((1, _H), lambda i: (0, 0)),
            pl.BlockSpec((1, _H), lambda i: (0, 0)),
            pl.BlockSpec((_H, _C), lambda i: (0, 0)),
            pl.BlockSpec((1, _C), lambda i: (0, 0)),
        ],
        out_specs=pl.BlockSpec((_BR, _C), lambda i: (i, 0)),
        out_shape=jax.ShapeDtypeStruct((_N, _C), jnp.float32),
    )


def kernel(x, edge_index, edge_type, W1, root1, b1, g1, be1,
           W2, root2, b2, g2, be2, W3, root3, b3, g3, be3, Wl, bl):
    src = edge_index[0]
    dst = edge_index[1]
    pad = _EPAD - _E
    seg = edge_type * _N + dst

    pk = jnp.concatenate(
        [src * 131072 + seg,
         jnp.full((pad,), _PADSEG, jnp.int32)]).reshape(_EPAD // _BLK, _BLK)

    agg = _make_sc_agg()
    layer1 = _make_layer(128, False)
    layer23 = _make_layer(_H, True)
    fin = _make_final()

    (cnt128,) = agg(jnp.ones((_NPAD, 128), jnp.float32), pk)
    cnt5 = cnt128.reshape(_R, _N, 128)
    xt = jnp.concatenate(
        [x, jnp.zeros((_NPAD - _N, 128), jnp.float32)])
    (a1,) = agg(xt, pk)

    z1, sums1 = layer1(xt, a1.reshape(_R, _N, 128), cnt5,
                       W1, root1, b1.reshape(1, _H))

    (a2,) = agg(z1, pk)
    z2, sums2 = layer23(z1, a2.reshape(_R, _N, 128), cnt5,
                        W2, root2, b2.reshape(1, _H),
                        sums1, g1.reshape(1, _H), be1.reshape(1, _H))

    (a3,) = agg(z2, pk)
    z3, sums3 = layer23(z2, a3.reshape(_R, _N, 128), cnt5,
                        W3, root3, b3.reshape(1, _H),
                        sums2, g2.reshape(1, _H), be2.reshape(1, _H))

    return fin(z3, sums3, g3.reshape(1, _H), be3.reshape(1, _H),
               Wl, bl.reshape(1, _C))
